# baseline (speedup 1.0000x reference)
import jax
import jax.numpy as jnp
from jax.experimental import pallas as pl
from jax.experimental.pallas import tpu as pltpu

T, H, F, E = 2048, 1024, 2048, 8
BT = 256
BF = 1024


def _moe_dense_kernel(logits_ref, x_ref, gate_ref, up_ref, down_ref, out_ref):
    e = pl.program_id(0)
    f = pl.program_id(1)
    t = pl.program_id(2)

    logits = logits_ref[...]
    probs = jax.nn.softmax(logits, axis=-1)
    m1 = jnp.max(probs, axis=-1, keepdims=True)
    cols = jax.lax.broadcasted_iota(jnp.int32, probs.shape, 1)
    i1 = jnp.argmax(probs, axis=-1)[:, None]
    masked = jnp.where(cols == i1, -jnp.inf, probs)
    m2 = jnp.max(masked, axis=-1, keepdims=True)
    i2 = jnp.argmax(masked, axis=-1)[:, None]
    denom = m1 + m2
    w_e = jnp.where(i1 == e, m1 / denom, 0.0) + jnp.where(i2 == e, m2 / denom, 0.0)

    x = x_ref[...]
    gate_w = gate_ref[0]
    up_w = up_ref[0]
    down_w = down_ref[0]
    dn = (((1,), (1,)), ((), ()))
    xb = x.astype(jnp.bfloat16)
    g = jax.lax.dot_general(xb, gate_w.astype(jnp.bfloat16), dn,
                            preferred_element_type=jnp.float32)
    u = jax.lax.dot_general(xb, up_w.astype(jnp.bfloat16), dn,
                            preferred_element_type=jnp.float32)
    h = (g * jax.lax.logistic(g)) * u
    o = jax.lax.dot_general(h.astype(jnp.bfloat16), down_w.astype(jnp.bfloat16),
                            dn, preferred_element_type=jnp.float32)
    contrib = o * w_e

    first = jnp.logical_and(e == 0, f == 0)

    @pl.when(first)
    def _init():
        out_ref[pl.ds(t * BT, BT), :] = contrib

    @pl.when(jnp.logical_not(first))
    def _acc():
        out_ref[pl.ds(t * BT, BT), :] += contrib


def kernel(x, router_logits, gate_up_proj, down_proj, top_k=2):
    gate_p = gate_up_proj[:, :F, :]
    up_p = gate_up_proj[:, F:, :]
    grid = (E, F // BF, T // BT)
    out = pl.pallas_call(
        _moe_dense_kernel,
        grid=grid,
        in_specs=[
            pl.BlockSpec((BT, E), lambda e, f, t: (t, 0)),
            pl.BlockSpec((BT, H), lambda e, f, t: (t, 0)),
            pl.BlockSpec((1, BF, H), lambda e, f, t: (e, f, 0)),
            pl.BlockSpec((1, BF, H), lambda e, f, t: (e, f, 0)),
            pl.BlockSpec((1, H, BF), lambda e, f, t: (e, 0, f)),
        ],
        out_specs=pl.BlockSpec((T, H), lambda e, f, t: (0, 0)),
        out_shape=jax.ShapeDtypeStruct((T, H), jnp.float32),
        compiler_params=pltpu.CompilerParams(
            dimension_semantics=("arbitrary", "arbitrary", "arbitrary"),
        ),
    )(router_logits, x, gate_p, up_p, down_proj)
    scale = jnp.asarray(top_k, jnp.float32) / jnp.float32(2)
    return out * scale


# speedup vs baseline: 2.0868x; 2.0868x over previous
import functools

import jax
import jax.numpy as jnp
from jax import lax
from jax.experimental import pallas as pl
from jax.experimental.pallas import tpu as pltpu
from jax.experimental.pallas import tpu_sc as plsc

T, H, F, E = 2048, 1024, 2048, 8
BR = 256
NB = (2 * T + E * BR) // BR
NP = NB * BR
BF = 1024
NF = F // BF
BT2 = 512

NC, NS = 2, 16
NW = NC * NS
CHUNK = T // NW


def _top2(logits):
    probs = jax.nn.softmax(logits, axis=-1)
    cols = lax.broadcasted_iota(jnp.int32, probs.shape, 1)
    i1 = jnp.argmax(probs, axis=-1)[:, None]
    m1 = jnp.max(probs, axis=-1, keepdims=True)
    masked = jnp.where(cols == i1, -jnp.inf, probs)
    i2 = jnp.argmax(masked, axis=-1)[:, None]
    m2 = jnp.max(masked, axis=-1, keepdims=True)
    denom = m1 + m2
    return i1, i2, m1 / denom, m2 / denom


def _route_kernel(logits_ref, p1_ref, p2_ref, be_ref):
    logits = logits_ref[...]
    i1, i2, _, _ = _top2(logits)
    cols = lax.broadcasted_iota(jnp.int32, (T, E), 1)
    mask1 = (cols == i1)
    mask2 = (cols == i2)
    cnt = mask1.astype(jnp.float32) + mask2.astype(jnp.float32)

    rows_i = lax.broadcasted_iota(jnp.int32, (T, T), 0)
    cols_i = lax.broadcasted_iota(jnp.int32, (T, T), 1)
    tril = (rows_i >= cols_i).astype(jnp.float32)
    cum = lax.dot_general(tril, cnt, (((1,), (0,)), ((), ())),
                          preferred_element_type=jnp.float32)
    cum_i = cum.astype(jnp.int32)

    counts = cum_i[T - 1:T, :]
    padded = ((counts + (BR - 1)) // BR) * BR
    ei = lax.broadcasted_iota(jnp.int32, (E, E), 0)
    ej = lax.broadcasted_iota(jnp.int32, (E, E), 1)
    triu = (ei <= ej).astype(jnp.float32)
    incl = lax.dot_general(padded.astype(jnp.float32), triu,
                           (((1,), (0,)), ((), ())),
                           preferred_element_type=jnp.float32).astype(jnp.int32)
    offs = incl - padded

    rank1 = jnp.sum(jnp.where(mask1, cum_i, 0), axis=1, keepdims=True) - 1
    rank2 = jnp.sum(jnp.where(mask2, cum_i, 0), axis=1, keepdims=True) - 1
    off1 = jnp.sum(jnp.where(mask1, offs, 0), axis=1, keepdims=True)
    off2 = jnp.sum(jnp.where(mask2, offs, 0), axis=1, keepdims=True)
    p1_ref[...] = (off1 + rank1)[:, 0]
    p2_ref[...] = (off2 + rank2)[:, 0]

    brow = lax.broadcasted_iota(jnp.int32, (NB, E), 0) * BR
    ge = (brow >= jnp.broadcast_to(offs, (NB, E))).astype(jnp.int32)
    be = jnp.sum(ge, axis=1, keepdims=True) - 1
    act = brow[:, :1] < jnp.broadcast_to(incl[:, E - 1:E], (NB, 1))
    be_ref[...] = jnp.broadcast_to(jnp.where(act, be, -1), (NB, 8))


def _route(router_logits):
    return pl.pallas_call(
        _route_kernel,
        out_shape=(
            jax.ShapeDtypeStruct((T,), jnp.int32),
            jax.ShapeDtypeStruct((T,), jnp.int32),
            jax.ShapeDtypeStruct((NB, 8), jnp.int32),
        ),
    )(router_logits)


def _make_scatter_rows():
    mesh = plsc.VectorSubcoreMesh(core_axis_name="c", subcore_axis_name="s")

    @functools.partial(
        pl.kernel, mesh=mesh,
        out_type=jax.ShapeDtypeStruct((NP, H), jnp.float32),
        scratch_types=[
            pltpu.VMEM((CHUNK,), jnp.int32),
            pltpu.VMEM((CHUNK, H), jnp.float32),
            pltpu.SemaphoreType.DMA,
        ],
    )
    def scatter_rows(x_hbm, p1_hbm, p2_hbm, xg_hbm, idx_v, rows_v, sem):
        wid = lax.axis_index("s") * NC + lax.axis_index("c")
        base = wid * CHUNK
        pltpu.sync_copy(x_hbm.at[pl.ds(base, CHUNK)], rows_v)
        pltpu.sync_copy(p1_hbm.at[pl.ds(base, CHUNK)], idx_v)
        pltpu.async_copy(rows_v, xg_hbm.at[idx_v], sem).wait()
        pltpu.sync_copy(p2_hbm.at[pl.ds(base, CHUNK)], idx_v)
        pltpu.async_copy(rows_v, xg_hbm.at[idx_v], sem).wait()

    return scatter_rows


_make_scatter_rows = functools.cache(_make_scatter_rows)


def _ffn_kernel(be_ref, xg_ref, gu_ref, down_ref, yg_ref):
    b = pl.program_id(0)

    @pl.when(be_ref[b, 0] >= 0)
    def _active():
        x = xg_ref[...]
        dn = (((1,), (1,)), ((), ()))
        o = jnp.zeros((BR, H), jnp.float32)
        for fh in range(NF):
            gate_w = gu_ref[0, pl.ds(fh * BF, BF), :]
            up_w = gu_ref[0, pl.ds(F + fh * BF, BF), :]
            down_w = down_ref[0, :, pl.ds(fh * BF, BF)]
            g = lax.dot_general(x, gate_w, dn, preferred_element_type=jnp.float32)
            u = lax.dot_general(x, up_w, dn, preferred_element_type=jnp.float32)
            h = (g * lax.logistic(g)) * u
            o += lax.dot_general(h, down_w, dn, preferred_element_type=jnp.float32)
        yg_ref[...] = o


def _ffn(be, xg, gate_up_proj, down_proj):
    grid_spec = pltpu.PrefetchScalarGridSpec(
        num_scalar_prefetch=1,
        grid=(NB,),
        in_specs=[
            pl.BlockSpec((BR, H), lambda b, be_ref: (b, 0)),
            pl.BlockSpec((1, 2 * F, H), lambda b, be_ref: (jnp.maximum(be_ref[b, 0], 0), 0, 0)),
            pl.BlockSpec((1, H, F), lambda b, be_ref: (jnp.maximum(be_ref[b, 0], 0), 0, 0)),
        ],
        out_specs=pl.BlockSpec((BR, H), lambda b, be_ref: (b, 0)),
    )
    return pl.pallas_call(
        _ffn_kernel,
        grid_spec=grid_spec,
        out_shape=jax.ShapeDtypeStruct((NP, H), jnp.float32),
        compiler_params=pltpu.CompilerParams(
            dimension_semantics=("arbitrary",),
        ),
    )(be, xg, gate_up_proj, down_proj)


def _make_gather_rows():
    mesh = plsc.VectorSubcoreMesh(core_axis_name="c", subcore_axis_name="s")

    @functools.partial(
        pl.kernel, mesh=mesh,
        out_type=(
            jax.ShapeDtypeStruct((T, H), jnp.float32),
            jax.ShapeDtypeStruct((T, H), jnp.float32),
        ),
        scratch_types=[
            pltpu.VMEM((CHUNK,), jnp.int32),
            pltpu.VMEM((CHUNK, H), jnp.float32),
            pltpu.SemaphoreType.DMA,
        ],
    )
    def gather_rows(yg_hbm, p1_hbm, p2_hbm, a_hbm, b_hbm, idx_v, rows_v, sem):
        wid = lax.axis_index("s") * NC + lax.axis_index("c")
        base = wid * CHUNK
        pltpu.sync_copy(p1_hbm.at[pl.ds(base, CHUNK)], idx_v)
        pltpu.async_copy(yg_hbm.at[idx_v], rows_v, sem).wait()
        pltpu.sync_copy(rows_v, a_hbm.at[pl.ds(base, CHUNK)])
        pltpu.sync_copy(p2_hbm.at[pl.ds(base, CHUNK)], idx_v)
        pltpu.async_copy(yg_hbm.at[idx_v], rows_v, sem).wait()
        pltpu.sync_copy(rows_v, b_hbm.at[pl.ds(base, CHUNK)])

    return gather_rows


_make_gather_rows = functools.cache(_make_gather_rows)


def _combine_kernel(logits_ref, a_ref, b_ref, out_ref):
    _, _, w1, w2 = _top2(logits_ref[...])
    out_ref[...] = w1 * a_ref[...] + w2 * b_ref[...]


def _combine(router_logits, a, b):
    return pl.pallas_call(
        _combine_kernel,
        grid=(T // BT2,),
        in_specs=[
            pl.BlockSpec((BT2, E), lambda t: (t, 0)),
            pl.BlockSpec((BT2, H), lambda t: (t, 0)),
            pl.BlockSpec((BT2, H), lambda t: (t, 0)),
        ],
        out_specs=pl.BlockSpec((BT2, H), lambda t: (t, 0)),
        out_shape=jax.ShapeDtypeStruct((T, H), jnp.float32),
    )(router_logits, a, b)


def kernel(x, router_logits, gate_up_proj, down_proj, top_k=2):
    p1, p2, be = _route(router_logits)
    xg = _make_scatter_rows()(x, p1, p2)
    yg = _ffn(be, xg, gate_up_proj, down_proj)
    a, b = _make_gather_rows()(yg, p1, p2)
    out = _combine(router_logits, a, b)
    scale = jnp.asarray(top_k, jnp.float32) / jnp.float32(2)
    return out * scale


# speedup vs baseline: 2.1357x; 1.0234x over previous
import functools

import jax
import jax.numpy as jnp
from jax import lax
from jax.experimental import pallas as pl
from jax.experimental.pallas import tpu as pltpu
from jax.experimental.pallas import tpu_sc as plsc

T, H, F, E = 2048, 1024, 2048, 8
BR = 256
NB = (2 * T + E * BR) // BR
NP = NB * BR
BF = 1024
NF = F // BF
BT2 = 512

NC, NS = 2, 16
NW = NC * NS
CHUNK = T // NW


def _top2(logits):
    probs = jax.nn.softmax(logits, axis=-1)
    cols = lax.broadcasted_iota(jnp.int32, probs.shape, 1)
    i1 = jnp.argmax(probs, axis=-1)[:, None]
    m1 = jnp.max(probs, axis=-1, keepdims=True)
    masked = jnp.where(cols == i1, -jnp.inf, probs)
    i2 = jnp.argmax(masked, axis=-1)[:, None]
    m2 = jnp.max(masked, axis=-1, keepdims=True)
    denom = m1 + m2
    return i1, i2, m1 / denom, m2 / denom


def _route_kernel(logits_ref, p1_ref, p2_ref, be_ref):
    logits = logits_ref[...]
    i1, i2, _, _ = _top2(logits)
    cols = lax.broadcasted_iota(jnp.int32, (T, E), 1)
    mask1 = (cols == i1)
    mask2 = (cols == i2)
    cnt = mask1.astype(jnp.float32) + mask2.astype(jnp.float32)

    rows_i = lax.broadcasted_iota(jnp.int32, (T, T), 0)
    cols_i = lax.broadcasted_iota(jnp.int32, (T, T), 1)
    tril = (rows_i >= cols_i).astype(jnp.float32)
    cum = lax.dot_general(tril, cnt, (((1,), (0,)), ((), ())),
                          preferred_element_type=jnp.float32)
    cum_i = cum.astype(jnp.int32)

    counts = cum_i[T - 1:T, :]
    padded = ((counts + (BR - 1)) // BR) * BR
    ei = lax.broadcasted_iota(jnp.int32, (E, E), 0)
    ej = lax.broadcasted_iota(jnp.int32, (E, E), 1)
    triu = (ei <= ej).astype(jnp.float32)
    incl = lax.dot_general(padded.astype(jnp.float32), triu,
                           (((1,), (0,)), ((), ())),
                           preferred_element_type=jnp.float32).astype(jnp.int32)
    offs = incl - padded

    rank1 = jnp.sum(jnp.where(mask1, cum_i, 0), axis=1, keepdims=True) - 1
    rank2 = jnp.sum(jnp.where(mask2, cum_i, 0), axis=1, keepdims=True) - 1
    off1 = jnp.sum(jnp.where(mask1, offs, 0), axis=1, keepdims=True)
    off2 = jnp.sum(jnp.where(mask2, offs, 0), axis=1, keepdims=True)
    p1_ref[...] = (off1 + rank1)[:, 0]
    p2_ref[...] = (off2 + rank2)[:, 0]

    brow = lax.broadcasted_iota(jnp.int32, (NB, E), 0) * BR
    ge = (brow >= jnp.broadcast_to(offs, (NB, E))).astype(jnp.int32)
    be = jnp.sum(ge, axis=1, keepdims=True) - 1
    act = brow[:, :1] < jnp.broadcast_to(incl[:, E - 1:E], (NB, 1))
    be_ref[...] = jnp.broadcast_to(jnp.where(act, be, -1), (NB, 8))


def _route(router_logits):
    return pl.pallas_call(
        _route_kernel,
        out_shape=(
            jax.ShapeDtypeStruct((T,), jnp.int32),
            jax.ShapeDtypeStruct((T,), jnp.int32),
            jax.ShapeDtypeStruct((NB, 8), jnp.int32),
        ),
    )(router_logits)


def _make_scatter_rows():
    mesh = plsc.VectorSubcoreMesh(core_axis_name="c", subcore_axis_name="s")

    @functools.partial(
        pl.kernel, mesh=mesh,
        out_type=jax.ShapeDtypeStruct((NP, H), jnp.float32),
        scratch_types=[
            pltpu.VMEM((CHUNK,), jnp.int32),
            pltpu.VMEM((CHUNK, H), jnp.float32),
            pltpu.SemaphoreType.DMA,
        ],
    )
    def scatter_rows(x_hbm, p1_hbm, p2_hbm, xg_hbm, idx_v, rows_v, sem):
        wid = lax.axis_index("s") * NC + lax.axis_index("c")
        base = wid * CHUNK
        pltpu.sync_copy(x_hbm.at[pl.ds(base, CHUNK)], rows_v)
        pltpu.sync_copy(p1_hbm.at[pl.ds(base, CHUNK)], idx_v)
        pltpu.async_copy(rows_v, xg_hbm.at[idx_v], sem).wait()
        pltpu.sync_copy(p2_hbm.at[pl.ds(base, CHUNK)], idx_v)
        pltpu.async_copy(rows_v, xg_hbm.at[idx_v], sem).wait()

    return scatter_rows


_make_scatter_rows = functools.cache(_make_scatter_rows)


def _ffn_kernel(be_ref, xg_ref, gu_ref, down_ref, yg_ref):
    b = pl.program_id(0)

    @pl.when(be_ref[b, 0] >= 0)
    def _active():
        x = xg_ref[...]
        dn = (((1,), (1,)), ((), ()))
        o = jnp.zeros((BR, H), jnp.float32)
        for fh in range(NF):
            gate_w = gu_ref[0, pl.ds(fh * BF, BF), :]
            up_w = gu_ref[0, pl.ds(F + fh * BF, BF), :]
            down_w = down_ref[0, :, pl.ds(fh * BF, BF)]
            g = lax.dot_general(x, gate_w, dn, preferred_element_type=jnp.float32)
            u = lax.dot_general(x, up_w, dn, preferred_element_type=jnp.float32)
            h = (g * lax.logistic(g)) * u
            o += lax.dot_general(h, down_w, dn, preferred_element_type=jnp.float32)
        yg_ref[...] = o


def _ffn(be, xg, gate_up_proj, down_proj):
    grid_spec = pltpu.PrefetchScalarGridSpec(
        num_scalar_prefetch=1,
        grid=(NB,),
        in_specs=[
            pl.BlockSpec((BR, H), lambda b, be_ref: (jnp.where(be_ref[b, 0] >= 0, b, 0), 0)),
            pl.BlockSpec((1, 2 * F, H), lambda b, be_ref: (jnp.where(be_ref[b, 0] >= 0, be_ref[b, 0], E - 1), 0, 0)),
            pl.BlockSpec((1, H, F), lambda b, be_ref: (jnp.where(be_ref[b, 0] >= 0, be_ref[b, 0], E - 1), 0, 0)),
        ],
        out_specs=pl.BlockSpec((BR, H), lambda b, be_ref: (b, 0)),
    )
    return pl.pallas_call(
        _ffn_kernel,
        grid_spec=grid_spec,
        out_shape=jax.ShapeDtypeStruct((NP, H), jnp.float32),
        compiler_params=pltpu.CompilerParams(
            dimension_semantics=("arbitrary",),
        ),
    )(be, xg, gate_up_proj, down_proj)


def _make_gather_rows():
    mesh = plsc.VectorSubcoreMesh(core_axis_name="c", subcore_axis_name="s")

    @functools.partial(
        pl.kernel, mesh=mesh,
        out_type=(
            jax.ShapeDtypeStruct((T, H), jnp.float32),
            jax.ShapeDtypeStruct((T, H), jnp.float32),
        ),
        scratch_types=[
            pltpu.VMEM((CHUNK,), jnp.int32),
            pltpu.VMEM((CHUNK, H), jnp.float32),
            pltpu.SemaphoreType.DMA,
        ],
    )
    def gather_rows(yg_hbm, p1_hbm, p2_hbm, a_hbm, b_hbm, idx_v, rows_v, sem):
        wid = lax.axis_index("s") * NC + lax.axis_index("c")
        base = wid * CHUNK
        pltpu.sync_copy(p1_hbm.at[pl.ds(base, CHUNK)], idx_v)
        pltpu.async_copy(yg_hbm.at[idx_v], rows_v, sem).wait()
        pltpu.sync_copy(rows_v, a_hbm.at[pl.ds(base, CHUNK)])
        pltpu.sync_copy(p2_hbm.at[pl.ds(base, CHUNK)], idx_v)
        pltpu.async_copy(yg_hbm.at[idx_v], rows_v, sem).wait()
        pltpu.sync_copy(rows_v, b_hbm.at[pl.ds(base, CHUNK)])

    return gather_rows


_make_gather_rows = functools.cache(_make_gather_rows)


def _combine_kernel(logits_ref, a_ref, b_ref, out_ref):
    _, _, w1, w2 = _top2(logits_ref[...])
    out_ref[...] = w1 * a_ref[...] + w2 * b_ref[...]


def _combine(router_logits, a, b):
    return pl.pallas_call(
        _combine_kernel,
        grid=(T // BT2,),
        in_specs=[
            pl.BlockSpec((BT2, E), lambda t: (t, 0)),
            pl.BlockSpec((BT2, H), lambda t: (t, 0)),
            pl.BlockSpec((BT2, H), lambda t: (t, 0)),
        ],
        out_specs=pl.BlockSpec((BT2, H), lambda t: (t, 0)),
        out_shape=jax.ShapeDtypeStruct((T, H), jnp.float32),
    )(router_logits, a, b)


def kernel(x, router_logits, gate_up_proj, down_proj, top_k=2):
    p1, p2, be = _route(router_logits)
    xg = _make_scatter_rows()(x, p1, p2)
    yg = _ffn(be, xg, gate_up_proj, down_proj)
    a, b = _make_gather_rows()(yg, p1, p2)
    out = _combine(router_logits, a, b)
    scale = jnp.asarray(top_k, jnp.float32) / jnp.float32(2)
    return out * scale


# speedup vs baseline: 2.2402x; 1.0489x over previous
import functools

import jax
import jax.numpy as jnp
from jax import lax
from jax.experimental import pallas as pl
from jax.experimental.pallas import tpu as pltpu
from jax.experimental.pallas import tpu_sc as plsc

T, H, F, E = 2048, 1024, 2048, 8
BR = 256
NB = (2 * T + E * BR) // BR
NP = NB * BR
BF = 1024
NF = F // BF
BT2 = 512

NC, NS = 2, 16
NW = NC * NS
CHUNK = T // NW


def _top2(logits):
    probs = jax.nn.softmax(logits, axis=-1)
    cols = lax.broadcasted_iota(jnp.int32, probs.shape, 1)
    i1 = jnp.argmax(probs, axis=-1)[:, None]
    m1 = jnp.max(probs, axis=-1, keepdims=True)
    masked = jnp.where(cols == i1, -jnp.inf, probs)
    i2 = jnp.argmax(masked, axis=-1)[:, None]
    m2 = jnp.max(masked, axis=-1, keepdims=True)
    denom = m1 + m2
    return i1, i2, m1 / denom, m2 / denom


def _route_kernel(logits_ref, p1_ref, p2_ref, be_ref):
    logits = logits_ref[...]
    i1, i2, _, _ = _top2(logits)
    cols = lax.broadcasted_iota(jnp.int32, (T, E), 1)
    mask1 = (cols == i1)
    mask2 = (cols == i2)
    cnt = mask1.astype(jnp.float32) + mask2.astype(jnp.float32)

    rows_i = lax.broadcasted_iota(jnp.int32, (T, T), 0)
    cols_i = lax.broadcasted_iota(jnp.int32, (T, T), 1)
    tril = (rows_i >= cols_i).astype(jnp.float32)
    cum = lax.dot_general(tril, cnt, (((1,), (0,)), ((), ())),
                          preferred_element_type=jnp.float32)
    cum_i = cum.astype(jnp.int32)

    counts = cum_i[T - 1:T, :]
    padded = ((counts + (BR - 1)) // BR) * BR
    ei = lax.broadcasted_iota(jnp.int32, (E, E), 0)
    ej = lax.broadcasted_iota(jnp.int32, (E, E), 1)
    triu = (ei <= ej).astype(jnp.float32)
    incl = lax.dot_general(padded.astype(jnp.float32), triu,
                           (((1,), (0,)), ((), ())),
                           preferred_element_type=jnp.float32).astype(jnp.int32)
    offs = incl - padded

    rank1 = jnp.sum(jnp.where(mask1, cum_i, 0), axis=1, keepdims=True) - 1
    rank2 = jnp.sum(jnp.where(mask2, cum_i, 0), axis=1, keepdims=True) - 1
    off1 = jnp.sum(jnp.where(mask1, offs, 0), axis=1, keepdims=True)
    off2 = jnp.sum(jnp.where(mask2, offs, 0), axis=1, keepdims=True)
    p1_ref[...] = (off1 + rank1)[:, 0]
    p2_ref[...] = (off2 + rank2)[:, 0]

    brow = lax.broadcasted_iota(jnp.int32, (NB, E), 0) * BR
    ge = (brow >= jnp.broadcast_to(offs, (NB, E))).astype(jnp.int32)
    be = jnp.sum(ge, axis=1, keepdims=True) - 1
    act = brow[:, :1] < jnp.broadcast_to(incl[:, E - 1:E], (NB, 1))
    be_ref[...] = jnp.broadcast_to(jnp.where(act, be, -1), (NB, 8))


def _route(router_logits):
    return pl.pallas_call(
        _route_kernel,
        out_shape=(
            jax.ShapeDtypeStruct((T,), jnp.int32),
            jax.ShapeDtypeStruct((T,), jnp.int32),
            jax.ShapeDtypeStruct((NB, 8), jnp.int32),
        ),
    )(router_logits)


def _make_scatter_rows():
    mesh = plsc.VectorSubcoreMesh(core_axis_name="c", subcore_axis_name="s")

    @functools.partial(
        pl.kernel, mesh=mesh,
        out_type=jax.ShapeDtypeStruct((NP, H), jnp.float32),
        scratch_types=[
            pltpu.VMEM((CHUNK,), jnp.int32),
            pltpu.VMEM((CHUNK, H), jnp.float32),
            pltpu.SemaphoreType.DMA,
        ],
    )
    def scatter_rows(x_hbm, p1_hbm, p2_hbm, xg_hbm, idx_v, rows_v, sem):
        wid = lax.axis_index("s") * NC + lax.axis_index("c")
        base = wid * CHUNK
        pltpu.sync_copy(x_hbm.at[pl.ds(base, CHUNK)], rows_v)
        pltpu.sync_copy(p1_hbm.at[pl.ds(base, CHUNK)], idx_v)
        pltpu.async_copy(rows_v, xg_hbm.at[idx_v], sem).wait()
        pltpu.sync_copy(p2_hbm.at[pl.ds(base, CHUNK)], idx_v)
        pltpu.async_copy(rows_v, xg_hbm.at[idx_v], sem).wait()

    return scatter_rows


_make_scatter_rows = functools.cache(_make_scatter_rows)


def _ffn_kernel(be_ref, xg_ref, gu_ref, down_ref, yg_ref):
    b = pl.program_id(0)

    @pl.when(be_ref[b, 0] >= 0)
    def _active():
        x = xg_ref[...]
        dn = (((1,), (1,)), ((), ()))
        o = jnp.zeros((BR, H), jnp.float32)
        for fh in range(NF):
            gate_w = gu_ref[0, pl.ds(fh * BF, BF), :]
            up_w = gu_ref[0, pl.ds(F + fh * BF, BF), :]
            down_w = down_ref[0, :, pl.ds(fh * BF, BF)]
            g = lax.dot_general(x, gate_w, dn, preferred_element_type=jnp.float32)
            u = lax.dot_general(x, up_w, dn, preferred_element_type=jnp.float32)
            h = (g * lax.logistic(g)) * u
            o += lax.dot_general(h, down_w, dn, preferred_element_type=jnp.float32)
        r = lax.bitcast_convert_type(o, jnp.uint32)
        rb = (r + jnp.uint32(0x7FFF) + ((r >> 16) & jnp.uint32(1))) >> 16
        packed = rb[:, :H // 2] | (rb[:, H // 2:] << 16)
        yg_ref[...] = lax.bitcast_convert_type(packed, jnp.int32)


def _ffn(be, xg, gate_up_proj, down_proj):
    grid_spec = pltpu.PrefetchScalarGridSpec(
        num_scalar_prefetch=1,
        grid=(NB,),
        in_specs=[
            pl.BlockSpec((BR, H), lambda b, be_ref: (jnp.where(be_ref[b, 0] >= 0, b, 0), 0)),
            pl.BlockSpec((1, 2 * F, H), lambda b, be_ref: (jnp.where(be_ref[b, 0] >= 0, be_ref[b, 0], E - 1), 0, 0)),
            pl.BlockSpec((1, H, F), lambda b, be_ref: (jnp.where(be_ref[b, 0] >= 0, be_ref[b, 0], E - 1), 0, 0)),
        ],
        out_specs=pl.BlockSpec((BR, H // 2), lambda b, be_ref: (b, 0)),
    )
    return pl.pallas_call(
        _ffn_kernel,
        grid_spec=grid_spec,
        out_shape=jax.ShapeDtypeStruct((NP, H // 2), jnp.int32),
        compiler_params=pltpu.CompilerParams(
            dimension_semantics=("arbitrary",),
        ),
    )(be, xg, gate_up_proj, down_proj)


def _make_gather_rows():
    mesh = plsc.VectorSubcoreMesh(core_axis_name="c", subcore_axis_name="s")

    @functools.partial(
        pl.kernel, mesh=mesh,
        out_type=(
            jax.ShapeDtypeStruct((T, H // 2), jnp.int32),
            jax.ShapeDtypeStruct((T, H // 2), jnp.int32),
        ),
        scratch_types=[
            pltpu.VMEM((CHUNK,), jnp.int32),
            pltpu.VMEM((CHUNK, H // 2), jnp.int32),
            pltpu.SemaphoreType.DMA,
        ],
    )
    def gather_rows(yg_hbm, p1_hbm, p2_hbm, a_hbm, b_hbm, idx_v, rows_v, sem):
        wid = lax.axis_index("s") * NC + lax.axis_index("c")
        base = wid * CHUNK
        pltpu.sync_copy(p1_hbm.at[pl.ds(base, CHUNK)], idx_v)
        pltpu.async_copy(yg_hbm.at[idx_v], rows_v, sem).wait()
        pltpu.sync_copy(rows_v, a_hbm.at[pl.ds(base, CHUNK)])
        pltpu.sync_copy(p2_hbm.at[pl.ds(base, CHUNK)], idx_v)
        pltpu.async_copy(yg_hbm.at[idx_v], rows_v, sem).wait()
        pltpu.sync_copy(rows_v, b_hbm.at[pl.ds(base, CHUNK)])

    return gather_rows


_make_gather_rows = functools.cache(_make_gather_rows)


def _unpack_bf16(packed_i32):
    p = lax.bitcast_convert_type(packed_i32, jnp.uint32)
    lo = lax.bitcast_convert_type((p & jnp.uint32(0xFFFF)) << 16, jnp.float32)
    hi = lax.bitcast_convert_type(p & jnp.uint32(0xFFFF0000), jnp.float32)
    return lo, hi


def _combine_kernel(logits_ref, a_ref, b_ref, out_ref):
    _, _, w1, w2 = _top2(logits_ref[...])
    a_lo, a_hi = _unpack_bf16(a_ref[...])
    b_lo, b_hi = _unpack_bf16(b_ref[...])
    out_ref[:, :H // 2] = w1 * a_lo + w2 * b_lo
    out_ref[:, H // 2:] = w1 * a_hi + w2 * b_hi


def _combine(router_logits, a, b):
    return pl.pallas_call(
        _combine_kernel,
        grid=(T // BT2,),
        in_specs=[
            pl.BlockSpec((BT2, E), lambda t: (t, 0)),
            pl.BlockSpec((BT2, H // 2), lambda t: (t, 0)),
            pl.BlockSpec((BT2, H // 2), lambda t: (t, 0)),
        ],
        out_specs=pl.BlockSpec((BT2, H), lambda t: (t, 0)),
        out_shape=jax.ShapeDtypeStruct((T, H), jnp.float32),
    )(router_logits, a, b)


def kernel(x, router_logits, gate_up_proj, down_proj, top_k=2):
    p1, p2, be = _route(router_logits)
    xg = _make_scatter_rows()(x, p1, p2)
    yg = _ffn(be, xg, gate_up_proj, down_proj)
    a, b = _make_gather_rows()(yg, p1, p2)
    out = _combine(router_logits, a, b)
    scale = jnp.asarray(top_k, jnp.float32) / jnp.float32(2)
    return out * scale
